# Initial kernel scaffold; baseline (speedup 1.0000x reference)
import jax
import jax.numpy as jnp
from jax.experimental import pallas as pl


def kernel(x, emb, sector_keys, memory_keys, knowledge, Wq, bq, Wo, bo, gamma, beta):
    raise NotImplementedError("write your pallas kernel here")


# speedup vs baseline: 10.1483x; 10.1483x over previous
import functools

import jax
import jax.numpy as jnp
from jax import lax
from jax.experimental import pallas as pl
from jax.experimental.pallas import tpu as pltpu
from jax.experimental.pallas import tpu_sc as plsc

_NEG_INF = float("-inf")
_BIG_I32 = 2**30


def _sc_row_gather(table, idx, rows_per_worker, chunk):
  n = idx.shape[0]
  d = table.shape[1]
  info = plsc.get_sparse_core_info()
  nc, ns = info.num_cores, info.num_subcores
  nw = nc * ns
  assert n == nw * rows_per_worker
  assert rows_per_worker % chunk == 0
  nchunks = rows_per_worker // chunk
  mesh = plsc.VectorSubcoreMesh(core_axis_name="c", subcore_axis_name="s")

  @functools.partial(
      pl.kernel,
      mesh=mesh,
      out_type=jax.ShapeDtypeStruct((n, d), table.dtype),
      scratch_types=[
          pltpu.VMEM((chunk,), jnp.int32),
          pltpu.VMEM((chunk, d), table.dtype),
          pltpu.SemaphoreType.DMA,
      ],
  )
  def k(table_hbm, idx_hbm, out_hbm, idx_v, rows_v, sem):
    wid = lax.axis_index("s") * nc + lax.axis_index("c")
    wbase = wid * rows_per_worker

    def body(i, carry):
      base = wbase + i * chunk
      pltpu.sync_copy(idx_hbm.at[pl.ds(base, chunk)], idx_v)
      pltpu.async_copy(table_hbm.at[idx_v], rows_v, sem).wait()
      pltpu.sync_copy(rows_v, out_hbm.at[pl.ds(base, chunk)])
      return carry

    lax.fori_loop(0, nchunks, body, 0)

  return k(table, idx)


def _k2_body(xe_ref, mk_ref, sk_ref, wq_ref, bq_ref,
             sd_ref, tq_ref, fidx_ref, tope_ref, zden_ref,
             *, k_sel, chunk_l):
  xe = xe_ref[...]
  mk = mk_ref[0]
  tb = xe.shape[0]
  m = mk.shape[0]
  nchunk = m // chunk_l

  sks = sk_ref[...]
  ss = lax.dot_general(xe, sks, (((1,), (1,)), ((), ())),
                       preferred_element_type=jnp.float32)
  ss = ss - jnp.max(ss, axis=1, keepdims=True)
  es = jnp.exp(ss)
  sd_ref[...] = es / jnp.sum(es, axis=1, keepdims=True)
  tq_ref[...] = lax.dot_general(xe, wq_ref[...], (((1,), (0,)), ((), ())),
                                preferred_element_type=jnp.float32) + bq_ref[...]

  scores = lax.dot_general(xe, mk, (((1,), (1,)), ((), ())),
                           preferred_element_type=jnp.float32)
  sc3 = scores.reshape(tb, nchunk, chunk_l)
  m1 = jnp.max(sc3, axis=2)
  rowmax = jnp.max(m1, axis=1, keepdims=True)
  zden_ref[0, 0, :] = jnp.sum(jnp.exp(scores - rowmax), axis=1)

  iota_c = lax.broadcasted_iota(jnp.int32, (tb, nchunk), 1)
  m1w = m1
  cids = []
  n_csel = min(k_sel, nchunk)
  for _ in range(n_csel):
    v = jnp.max(m1w, axis=1, keepdims=True)
    cid = jnp.min(jnp.where(m1w == v, iota_c, _BIG_I32), axis=1)
    cids.append(cid)
    m1w = jnp.where(iota_c == cid[:, None], _NEG_INF, m1w)

  comp_parts = []
  fidx_parts = []
  iota_l = lax.broadcasted_iota(jnp.int32, (tb, chunk_l), 1)
  for cid in cids:
    onehot = (iota_c == cid[:, None]).astype(jnp.float32)
    part = jnp.sum(sc3 * onehot[:, :, None], axis=1)
    comp_parts.append(part)
    fidx_parts.append(cid[:, None] * chunk_l + iota_l)
  comp = jnp.concatenate(comp_parts, axis=1)
  flatc = jnp.concatenate(fidx_parts, axis=1)

  vals = []
  fidxs = []
  cw = comp
  for _ in range(k_sel):
    v = jnp.max(cw, axis=1, keepdims=True)
    fi = jnp.min(jnp.where(cw == v, flatc, _BIG_I32), axis=1)
    vals.append(v[:, 0])
    fidxs.append(fi)
    cw = jnp.where(flatc == fi[:, None], _NEG_INF, cw)

  vstack = jnp.stack(vals, axis=1)
  fidx_ref[0] = jnp.stack(fidxs, axis=1)
  tope_ref[0] = jnp.exp(vstack - rowmax)


def _k2_call(x_emb, memory_keys, sector_keys, wq, bq, *, k_sel, tb, chunk_l,
             interpret=False):
  s_tot, d = x_emb.shape
  ns, m, _ = memory_keys.shape
  nt = s_tot // tb
  grid = (ns, nt)
  out_shapes = [
      jax.ShapeDtypeStruct((s_tot, ns), jnp.float32),
      jax.ShapeDtypeStruct((s_tot, k_sel * 4), jnp.float32),
      jax.ShapeDtypeStruct((ns, s_tot, k_sel), jnp.int32),
      jax.ShapeDtypeStruct((ns, s_tot, k_sel), jnp.float32),
      jax.ShapeDtypeStruct((ns, 1, s_tot), jnp.float32),
  ]
  kd = wq.shape[1]
  out_shapes[1] = jax.ShapeDtypeStruct((s_tot, kd), jnp.float32)
  return pl.pallas_call(
      functools.partial(_k2_body, k_sel=k_sel, chunk_l=chunk_l),
      grid=grid,
      in_specs=[
          pl.BlockSpec((tb, d), lambda s, t: (t, 0)),
          pl.BlockSpec((1, m, d), lambda s, t: (s, 0, 0)),
          pl.BlockSpec((ns, d), lambda s, t: (0, 0)),
          pl.BlockSpec((d, kd), lambda s, t: (0, 0)),
          pl.BlockSpec((1, kd), lambda s, t: (0, 0)),
      ],
      out_specs=[
          pl.BlockSpec((tb, ns), lambda s, t: (t, 0)),
          pl.BlockSpec((tb, kd), lambda s, t: (t, 0)),
          pl.BlockSpec((1, tb, k_sel), lambda s, t: (s, t, 0)),
          pl.BlockSpec((1, tb, k_sel), lambda s, t: (s, t, 0)),
          pl.BlockSpec((1, 1, tb), lambda s, t: (s, 0, t)),
      ],
      out_shape=out_shapes,
      interpret=interpret,
  )(x_emb, memory_keys, sector_keys, wq, bq.reshape(1, kd))


def _k4_body(staged_ref, tope_ref, zden_ref, sd_ref, tq_ref, xe_ref,
             wo_ref, bo_ref, g_ref, b_ref, out_ref, acc_ref,
             *, k_sel, tb, ns, kd, vd):
  s = pl.program_id(1)
  t = pl.program_id(0)
  tpos = t * tb

  e_all = tope_ref[s]
  z_all = zden_ref[s, 0, :]
  sums_all = jnp.sum(e_all, axis=1) / z_all
  flag = jnp.all(sums_all > 1e-9)

  e_blk = tope_ref[s, pl.ds(tpos, tb), :]
  z_blk = zden_ref[s, 0, pl.ds(tpos, tb)]
  sum_e = jnp.sum(e_blk, axis=1)
  wnorm = jnp.where(flag, e_blk / sum_e[:, None], e_blk / z_blk[:, None])
  sd_blk = sd_ref[pl.ds(tpos, tb), :]
  iota_s = lax.broadcasted_iota(jnp.int32, sd_blk.shape, 1)
  sd_s = jnp.sum(jnp.where(iota_s == s, sd_blk, 0.0), axis=1)
  wtilde = wnorm * sd_s[:, None]

  ent = staged_ref[0].reshape(tb, k_sel, kd * vd)
  contrib = jnp.sum(ent * wtilde[:, :, None], axis=1)

  @pl.when(s == 0)
  def _():
    acc_ref[...] = contrib

  @pl.when(s > 0)
  def _():
    acc_ref[...] = acc_ref[...] + contrib

  @pl.when(s == ns - 1)
  def _():
    a3 = acc_ref[...].reshape(tb, kd, vd)
    q = tq_ref[...]
    ctx = jnp.sum(a3 * q[:, :, None], axis=1)
    proj = lax.dot_general(ctx, wo_ref[...], (((1,), (0,)), ((), ())),
                           preferred_element_type=jnp.float32) + bo_ref[...]
    h = xe_ref[...] + proj
    mean = jnp.mean(h, axis=1, keepdims=True)
    var = jnp.mean((h - mean) * (h - mean), axis=1, keepdims=True)
    out_ref[...] = (h - mean) * lax.rsqrt(var + 1e-5) * g_ref[...] + b_ref[...]


def _k4_call(staged, tope, zden, sd, tq, x_emb, wo, bo, gamma, beta,
             *, k_sel, tb, interpret=False):
  ns, s_tot, kcols = staged.shape
  d = x_emb.shape[1]
  kd = tq.shape[1]
  vd = wo.shape[0]
  nt = s_tot // tb
  grid = (nt, ns)
  full = lambda *shape: pl.BlockSpec(shape, lambda t, s: (0,) * len(shape))
  return pl.pallas_call(
      functools.partial(_k4_body, k_sel=k_sel, tb=tb, ns=ns, kd=kd, vd=vd),
      grid=grid,
      in_specs=[
          pl.BlockSpec((1, tb, kcols), lambda t, s: (s, t, 0)),
          full(ns, s_tot, k_sel),
          full(ns, 1, s_tot),
          full(s_tot, ns),
          pl.BlockSpec((tb, kd), lambda t, s: (t, 0)),
          pl.BlockSpec((tb, d), lambda t, s: (t, 0)),
          full(vd, d),
          full(1, d),
          full(1, d),
          full(1, d),
      ],
      out_specs=pl.BlockSpec((tb, d), lambda t, s: (t, 0)),
      out_shape=jax.ShapeDtypeStruct((s_tot, d), jnp.float32),
      scratch_shapes=[pltpu.VMEM((tb, kd * vd), jnp.float32)],
      interpret=interpret,
  )(staged, tope, zden, sd, tq, x_emb, wo, bo.reshape(1, d),
    gamma.reshape(1, d), beta.reshape(1, d))


def kernel(x, emb, sector_keys, memory_keys, knowledge, Wq, bq, Wo, bo,
           gamma, beta):
  b, s_tot = x.shape
  ns, m, kd, vd = knowledge.shape
  d = emb.shape[1]
  k_sel = 8

  xf = x.reshape(b * s_tot)
  x_emb = _sc_row_gather(emb, xf, rows_per_worker=(b * s_tot) // 32, chunk=64)

  sd, tq, fidx, tope, zden = _k2_call(
      x_emb, memory_keys, sector_keys, Wq, bq,
      k_sel=k_sel, tb=128, chunk_l=128)

  gidx = (fidx + (jnp.arange(ns, dtype=jnp.int32) * m)[:, None, None]
          ).reshape(-1)
  table = knowledge.reshape(ns * m, kd * vd)
  nrows = gidx.shape[0]
  staged = _sc_row_gather(table, gidx, rows_per_worker=nrows // 32, chunk=32)
  staged = staged.reshape(ns, s_tot, k_sel * kd * vd)

  out = _k4_call(staged, tope, zden, sd, tq, x_emb, Wo, bo, gamma, beta,
                 k_sel=k_sel, tb=256)
  return out.reshape(b, s_tot, d), sd.reshape(b, s_tot, ns)


# speedup vs baseline: 18.9731x; 1.8696x over previous
import functools

import jax
import jax.numpy as jnp
from jax import lax
from jax.experimental import pallas as pl
from jax.experimental.pallas import tpu as pltpu
from jax.experimental.pallas import tpu_sc as plsc

_NEG_INF = float("-inf")
_BIG_I32 = 2**30


def _sc_row_gather(table, idx, rows_per_worker, chunk):
  n = idx.shape[0]
  d = table.shape[1]
  info = plsc.get_sparse_core_info()
  nc, ns = info.num_cores, info.num_subcores
  nw = nc * ns
  assert n == nw * rows_per_worker
  assert rows_per_worker % chunk == 0
  nchunks = rows_per_worker // chunk
  assert nchunks % 2 == 0 and nchunks >= 2
  mesh = plsc.VectorSubcoreMesh(core_axis_name="c", subcore_axis_name="s")

  @functools.partial(
      pl.kernel,
      mesh=mesh,
      out_type=jax.ShapeDtypeStruct((n, d), table.dtype),
      scratch_types=[
          pltpu.VMEM((2, chunk), jnp.int32),
          pltpu.VMEM((2, chunk, d), table.dtype),
          pltpu.SemaphoreType.DMA((2,)),
          pltpu.SemaphoreType.DMA((2,)),
      ],
  )
  def k(table_hbm, idx_hbm, out_hbm, idx_v, rows_v, gsem, wsem):
    wid = lax.axis_index("s") * nc + lax.axis_index("c")
    wbase = wid * rows_per_worker

    def start_gather(i, p):
      base = wbase + i * chunk
      pltpu.sync_copy(idx_hbm.at[pl.ds(base, chunk)], idx_v.at[p])
      pltpu.async_copy(table_hbm.at[idx_v.at[p]], rows_v.at[p], gsem.at[p])

    def wait_gather(p):
      pltpu.make_async_copy(
          table_hbm.at[idx_v.at[p]], rows_v.at[p], gsem.at[p]).wait()

    def start_write(i, p):
      base = wbase + i * chunk
      pltpu.async_copy(rows_v.at[p], out_hbm.at[pl.ds(base, chunk)],
                       wsem.at[p])

    def wait_write(i, p):
      base = wbase + i * chunk
      pltpu.make_async_copy(
          rows_v.at[p], out_hbm.at[pl.ds(base, chunk)], wsem.at[p]).wait()

    start_gather(0, 0)

    @pl.loop(0, nchunks, step=2)
    def _(i2):
      for b in range(2):
        i = i2 + b
        p = b
        q = 1 - b

        @pl.when(i + 1 < nchunks)
        def _():
          @pl.when(i >= 1)
          def _():
            wait_write(i - 1, q)
          start_gather(i + 1, q)

        wait_gather(p)
        start_write(i, p)

    wait_write(nchunks - 2, 0)
    wait_write(nchunks - 1, 1)

  return k(table, idx)


def _k2_body(xe_ref, mk_ref, sk_ref, wq_ref, bq_ref,
             sd_ref, tq_ref, fidx_ref, tope_ref,
             *, k_sel, chunk_l):
  xe = xe_ref[...]
  mk = mk_ref[0]
  tb = xe.shape[0]
  m = mk.shape[0]
  nchunk = m // chunk_l

  sks = sk_ref[...]
  ss = lax.dot_general(xe, sks, (((1,), (1,)), ((), ())),
                       preferred_element_type=jnp.float32)
  ss = ss - jnp.max(ss, axis=1, keepdims=True)
  es = jnp.exp(ss)
  sd_ref[...] = es / jnp.sum(es, axis=1, keepdims=True)
  tq_ref[...] = lax.dot_general(xe, wq_ref[...], (((1,), (0,)), ((), ())),
                                preferred_element_type=jnp.float32) + bq_ref[...]

  scores = lax.dot_general(xe, mk, (((1,), (1,)), ((), ())),
                           preferred_element_type=jnp.float32)
  sc3 = scores.reshape(tb, nchunk, chunk_l)
  m1 = jnp.max(sc3, axis=2)
  rowmax = jnp.max(m1, axis=1, keepdims=True)

  big_f = float(2**24)
  iota_c = lax.broadcasted_iota(jnp.int32, (tb, nchunk), 1).astype(jnp.float32)
  m1w = m1
  cids = []
  n_csel = min(k_sel, nchunk)
  for _ in range(n_csel):
    v = jnp.max(m1w, axis=1, keepdims=True)
    cid = jnp.min(jnp.where(m1w == v, iota_c, big_f), axis=1)
    cids.append(cid)
    m1w = jnp.where(iota_c == cid[:, None], _NEG_INF, m1w)

  cids_arr = jnp.stack(cids, axis=1)
  iota_c3 = lax.broadcasted_iota(jnp.int32, (tb, n_csel, nchunk),
                                 2).astype(jnp.float32)
  onehot8 = (cids_arr[:, :, None] == iota_c3
             ).astype(jnp.float32)
  cw = lax.dot_general(onehot8, sc3, (((2,), (1,)), ((0,), (0,))),
                       preferred_element_type=jnp.float32)
  iota_l3 = lax.broadcasted_iota(jnp.int32, (tb, n_csel, chunk_l),
                                 2).astype(jnp.float32)
  flatc = cids_arr[:, :, None] * float(chunk_l) + iota_l3

  vals = []
  fidxs = []
  for _ in range(k_sel):
    v = jnp.max(jnp.max(cw, axis=2), axis=1)
    eq = cw == v[:, None, None]
    fi = jnp.min(jnp.min(jnp.where(eq, flatc, big_f), axis=2), axis=1)
    vals.append(v)
    fidxs.append(fi)
    cw = jnp.where(flatc == fi[:, None, None], _NEG_INF, cw)

  vstack = jnp.stack(vals, axis=1)
  fidx_ref[0] = jnp.stack(fidxs, axis=1).astype(jnp.int32)
  tope_ref[0] = jnp.exp(vstack - rowmax)


def _k2_call(x_emb, memory_keys, sector_keys, wq, bq, *, k_sel, tb, chunk_l,
             interpret=False):
  s_tot, d = x_emb.shape
  ns, m, _ = memory_keys.shape
  nt = s_tot // tb
  grid = (ns, nt)
  kd = wq.shape[1]
  out_shapes = [
      jax.ShapeDtypeStruct((s_tot, ns), jnp.float32),
      jax.ShapeDtypeStruct((s_tot, kd), jnp.float32),
      jax.ShapeDtypeStruct((ns, s_tot, k_sel), jnp.int32),
      jax.ShapeDtypeStruct((ns, s_tot, k_sel), jnp.float32),
  ]
  return pl.pallas_call(
      functools.partial(_k2_body, k_sel=k_sel, chunk_l=chunk_l),
      grid=grid,
      in_specs=[
          pl.BlockSpec((tb, d), lambda s, t: (t, 0)),
          pl.BlockSpec((1, m, d), lambda s, t: (s, 0, 0)),
          pl.BlockSpec((ns, d), lambda s, t: (0, 0)),
          pl.BlockSpec((d, kd), lambda s, t: (0, 0)),
          pl.BlockSpec((1, kd), lambda s, t: (0, 0)),
      ],
      out_specs=[
          pl.BlockSpec((tb, ns), lambda s, t: (t, 0)),
          pl.BlockSpec((tb, kd), lambda s, t: (t, 0)),
          pl.BlockSpec((1, tb, k_sel), lambda s, t: (s, t, 0)),
          pl.BlockSpec((1, tb, k_sel), lambda s, t: (s, t, 0)),
      ],
      out_shape=out_shapes,
      interpret=interpret,
  )(x_emb, memory_keys, sector_keys, wq, bq.reshape(1, kd))


def _k4_body(staged_ref, tope_ref, sd_ref, tq_ref, xe_ref,
             wo_ref, bo_ref, g_ref, b_ref, out_ref, acc_ref,
             *, k_sel, tb, ns, kd, vd):
  s = pl.program_id(1)
  t = pl.program_id(0)
  tpos = t * tb

  e_blk = tope_ref[s, pl.ds(tpos, tb), :]
  sum_e = jnp.sum(e_blk, axis=1)
  wnorm = e_blk / sum_e[:, None]
  sd_blk = sd_ref[pl.ds(tpos, tb), :]
  iota_s = lax.broadcasted_iota(jnp.int32, sd_blk.shape, 1)
  sd_s = jnp.sum(jnp.where(iota_s == s, sd_blk, 0.0), axis=1)
  wtilde = wnorm * sd_s[:, None]

  ent = staged_ref[0]
  scaled = ent * wtilde[:, :, None]
  @pl.when(s == 0)
  def _():
    acc_ref[...] = scaled

  @pl.when(s > 0)
  def _():
    acc_ref[...] = acc_ref[...] + scaled

  @pl.when(s == ns - 1)
  def _():
    q = tq_ref[...]
    io_l = lax.broadcasted_iota(jnp.int32, (kd, kd * vd), 1)
    io_d = lax.broadcasted_iota(jnp.int32, (kd, kd * vd), 0)
    spread = (io_l // vd == io_d).astype(jnp.float32)
    qrep = lax.dot_general(q, spread, (((1,), (0,)), ((), ())),
                           preferred_element_type=jnp.float32)
    prod = acc_ref[...] * qrep[:, None, :]
    w = kd * vd
    while w > vd:
      half = w // 2
      prod = prod[:, :, :half] + prod[:, :, half:w]
      w = half
    ctx = jnp.sum(prod, axis=1)
    proj = lax.dot_general(ctx, wo_ref[...], (((1,), (0,)), ((), ())),
                           preferred_element_type=jnp.float32) + bo_ref[...]
    h = xe_ref[...] + proj
    mean = jnp.mean(h, axis=1, keepdims=True)
    var = jnp.mean((h - mean) * (h - mean), axis=1, keepdims=True)
    out_ref[...] = (h - mean) * lax.rsqrt(var + 1e-5) * g_ref[...] + b_ref[...]


def _k4_call(staged, tope, sd, tq, x_emb, wo, bo, gamma, beta,
             *, k_sel, tb, interpret=False):
  nrows, kvcols = staged.shape
  ns, s_tot, _ = tope.shape
  d = x_emb.shape[1]
  kd = tq.shape[1]
  vd = wo.shape[0]
  nt = s_tot // tb
  grid = (nt, ns)
  staged4 = staged.reshape(ns * nt, tb, k_sel, kvcols)
  full = lambda *shape: pl.BlockSpec(shape, lambda t, s: (0,) * len(shape))
  return pl.pallas_call(
      functools.partial(_k4_body, k_sel=k_sel, tb=tb, ns=ns, kd=kd, vd=vd),
      grid=grid,
      in_specs=[
          pl.BlockSpec((1, tb, k_sel, kvcols), lambda t, s: (s * nt + t, 0, 0, 0)),
          full(ns, s_tot, k_sel),
          full(s_tot, ns),
          pl.BlockSpec((tb, kd), lambda t, s: (t, 0)),
          pl.BlockSpec((tb, d), lambda t, s: (t, 0)),
          full(vd, d),
          full(1, d),
          full(1, d),
          full(1, d),
      ],
      out_specs=pl.BlockSpec((tb, d), lambda t, s: (t, 0)),
      out_shape=jax.ShapeDtypeStruct((s_tot, d), jnp.float32),
      scratch_shapes=[pltpu.VMEM((tb, k_sel, kvcols), jnp.float32)],
      interpret=interpret,
  )(staged4, tope, sd, tq, x_emb, wo, bo.reshape(1, d),
    gamma.reshape(1, d), beta.reshape(1, d))


def kernel(x, emb, sector_keys, memory_keys, knowledge, Wq, bq, Wo, bo,
           gamma, beta):
  b, s_tot = x.shape
  ns, m, kd, vd = knowledge.shape
  d = emb.shape[1]
  k_sel = 8

  xf = x.reshape(b * s_tot)
  x_emb = _sc_row_gather(emb, xf, rows_per_worker=(b * s_tot) // 32, chunk=32)

  sd, tq, fidx, tope = _k2_call(
      x_emb, memory_keys, sector_keys, Wq, bq,
      k_sel=k_sel, tb=128, chunk_l=128)

  gidx = (fidx + (jnp.arange(ns, dtype=jnp.int32) * m)[:, None, None]
          ).reshape(-1)
  table = knowledge.reshape(ns * m, kd * vd)
  nrows = gidx.shape[0]
  staged = _sc_row_gather(table, gidx, rows_per_worker=nrows // 32, chunk=32)

  out = _k4_call(staged, tope, sd, tq, x_emb, Wo, bo, gamma, beta,
                 k_sel=k_sel, tb=256)
  return out.reshape(b, s_tot, d), sd.reshape(b, s_tot, ns)


# speedup vs baseline: 19.2529x; 1.0147x over previous
import functools

import jax
import jax.numpy as jnp
from jax import lax
from jax.experimental import pallas as pl
from jax.experimental.pallas import tpu as pltpu
from jax.experimental.pallas import tpu_sc as plsc

_NEG_INF = float("-inf")
_BIG_I32 = 2**30


def _sc_row_gather(table, idx, rows_per_worker, chunk):
  n = idx.shape[0]
  row_shape = table.shape[1:]
  info = plsc.get_sparse_core_info()
  nc, ns = info.num_cores, info.num_subcores
  nw = nc * ns
  assert n == nw * rows_per_worker
  assert rows_per_worker % chunk == 0
  nchunks = rows_per_worker // chunk
  assert nchunks % 2 == 0 and nchunks >= 2
  mesh = plsc.VectorSubcoreMesh(core_axis_name="c", subcore_axis_name="s")

  @functools.partial(
      pl.kernel,
      mesh=mesh,
      out_type=jax.ShapeDtypeStruct((n,) + row_shape, table.dtype),
      scratch_types=[
          pltpu.VMEM((2, chunk), jnp.int32),
          pltpu.VMEM((2, chunk) + row_shape, table.dtype),
          pltpu.SemaphoreType.DMA((2,)),
          pltpu.SemaphoreType.DMA((2,)),
      ],
  )
  def k(table_hbm, idx_hbm, out_hbm, idx_v, rows_v, gsem, wsem):
    wid = lax.axis_index("s") * nc + lax.axis_index("c")
    wbase = wid * rows_per_worker

    def start_gather(i, p):
      base = wbase + i * chunk
      pltpu.sync_copy(idx_hbm.at[pl.ds(base, chunk)], idx_v.at[p])
      pltpu.async_copy(table_hbm.at[idx_v.at[p]], rows_v.at[p], gsem.at[p])

    def wait_gather(p):
      pltpu.make_async_copy(
          table_hbm.at[idx_v.at[p]], rows_v.at[p], gsem.at[p]).wait()

    def start_write(i, p):
      base = wbase + i * chunk
      pltpu.async_copy(rows_v.at[p], out_hbm.at[pl.ds(base, chunk)],
                       wsem.at[p])

    def wait_write(i, p):
      base = wbase + i * chunk
      pltpu.make_async_copy(
          rows_v.at[p], out_hbm.at[pl.ds(base, chunk)], wsem.at[p]).wait()

    start_gather(0, 0)

    @pl.loop(0, nchunks, step=2)
    def _(i2):
      for b in range(2):
        i = i2 + b
        p = b
        q = 1 - b

        @pl.when(i + 1 < nchunks)
        def _():
          @pl.when(i >= 1)
          def _():
            wait_write(i - 1, q)
          start_gather(i + 1, q)

        wait_gather(p)
        start_write(i, p)

    wait_write(nchunks - 2, 0)
    wait_write(nchunks - 1, 1)

  return k(table, idx)


def _k2_body(xe_ref, mk_ref, sk_ref, wq_ref, bq_ref,
             sd_ref, tq_ref, fidx_ref, tope_ref,
             *, k_sel, chunk_l):
  xe = xe_ref[...]
  mk = mk_ref[0]
  tb = xe.shape[0]
  m = mk.shape[0]
  nchunk = m // chunk_l

  sks = sk_ref[...]
  ss = lax.dot_general(xe, sks, (((1,), (1,)), ((), ())),
                       preferred_element_type=jnp.float32)
  ss = ss - jnp.max(ss, axis=1, keepdims=True)
  es = jnp.exp(ss)
  sd_ref[...] = es / jnp.sum(es, axis=1, keepdims=True)
  tq_ref[...] = lax.dot_general(xe, wq_ref[...], (((1,), (0,)), ((), ())),
                                preferred_element_type=jnp.float32) + bq_ref[...]

  scores = lax.dot_general(xe, mk, (((1,), (1,)), ((), ())),
                           preferred_element_type=jnp.float32)
  sc3 = scores.reshape(tb, nchunk, chunk_l)
  m1 = jnp.max(sc3, axis=2)
  rowmax = jnp.max(m1, axis=1, keepdims=True)

  big_f = float(2**24)
  iota_c = lax.broadcasted_iota(jnp.int32, (tb, nchunk), 1).astype(jnp.float32)
  m1w = m1
  cids = []
  n_csel = min(k_sel, nchunk)
  for _ in range(n_csel):
    v = jnp.max(m1w, axis=1, keepdims=True)
    cid = jnp.min(jnp.where(m1w == v, iota_c, big_f), axis=1)
    cids.append(cid)
    m1w = jnp.where(iota_c == cid[:, None], _NEG_INF, m1w)

  cids_arr = jnp.stack(cids, axis=1)
  iota_c3 = lax.broadcasted_iota(jnp.int32, (tb, n_csel, nchunk),
                                 2).astype(jnp.float32)
  onehot8 = (cids_arr[:, :, None] == iota_c3
             ).astype(jnp.float32)
  cw = lax.dot_general(onehot8, sc3, (((2,), (1,)), ((0,), (0,))),
                       preferred_element_type=jnp.float32)
  iota_l3 = lax.broadcasted_iota(jnp.int32, (tb, n_csel, chunk_l),
                                 2).astype(jnp.float32)
  flatc = cids_arr[:, :, None] * float(chunk_l) + iota_l3

  vals = []
  fidxs = []
  for _ in range(k_sel):
    v = jnp.max(jnp.max(cw, axis=2), axis=1)
    eq = cw == v[:, None, None]
    fi = jnp.min(jnp.min(jnp.where(eq, flatc, big_f), axis=2), axis=1)
    vals.append(v)
    fidxs.append(fi)
    cw = jnp.where(flatc == fi[:, None, None], _NEG_INF, cw)

  vstack = jnp.stack(vals, axis=1)
  fidx_ref[0] = jnp.stack(fidxs, axis=1).astype(jnp.int32)
  tope_ref[0] = jnp.exp(vstack - rowmax)


def _k2_call(x_emb, memory_keys, sector_keys, wq, bq, *, k_sel, tb, chunk_l,
             interpret=False):
  s_tot, d = x_emb.shape
  nsp, m, _ = memory_keys.shape
  ns_all = sector_keys.shape[0]
  nt = s_tot // tb
  grid = (nsp, nt)
  kd = wq.shape[1]
  out_shapes = [
      jax.ShapeDtypeStruct((s_tot, ns_all), jnp.float32),
      jax.ShapeDtypeStruct((s_tot, kd), jnp.float32),
      jax.ShapeDtypeStruct((nsp, s_tot, k_sel), jnp.int32),
      jax.ShapeDtypeStruct((nsp, s_tot, k_sel), jnp.float32),
  ]
  return pl.pallas_call(
      functools.partial(_k2_body, k_sel=k_sel, chunk_l=chunk_l),
      grid=grid,
      in_specs=[
          pl.BlockSpec((tb, d), lambda s, t: (t, 0)),
          pl.BlockSpec((1, m, d), lambda s, t: (s, 0, 0)),
          pl.BlockSpec((ns_all, d), lambda s, t: (0, 0)),
          pl.BlockSpec((d, kd), lambda s, t: (0, 0)),
          pl.BlockSpec((1, kd), lambda s, t: (0, 0)),
      ],
      out_specs=[
          pl.BlockSpec((tb, ns_all), lambda s, t: (t, 0)),
          pl.BlockSpec((tb, kd), lambda s, t: (t, 0)),
          pl.BlockSpec((1, tb, k_sel), lambda s, t: (s, t, 0)),
          pl.BlockSpec((1, tb, k_sel), lambda s, t: (s, t, 0)),
      ],
      out_shape=out_shapes,
      interpret=interpret,
  )(x_emb, memory_keys, sector_keys, wq, bq.reshape(1, kd))


def _k4_body(staged_ref, tope_ref, sd_ref, tq_ref, ctx_ref,
             *, k_sel, tb, kd, vd, s_off):
  s = pl.program_id(1)
  t = pl.program_id(0)
  tpos = t * tb

  e_blk = tope_ref[s, pl.ds(tpos, tb), :]
  sum_e = jnp.sum(e_blk, axis=1)
  wnorm = e_blk / sum_e[:, None]
  sd_blk = sd_ref[pl.ds(tpos, tb), :]
  iota_s = lax.broadcasted_iota(jnp.int32, sd_blk.shape, 1)
  sd_s = jnp.sum(jnp.where(iota_s == s + s_off, sd_blk, 0.0), axis=1)
  wtilde = wnorm * sd_s[:, None]

  ent = staged_ref[0]
  sl = ent.shape[2]
  ll = ent.shape[3]
  scaled = ent * wtilde[:, :, None, None]

  q = tq_ref[...]
  io_l = lax.broadcasted_iota(jnp.int32, (kd, kd * vd), 1)
  io_d = lax.broadcasted_iota(jnp.int32, (kd, kd * vd), 0)
  spread = (io_l // vd == io_d).astype(jnp.float32)
  qrep = lax.dot_general(q, spread, (((1,), (0,)), ((), ())),
                         preferred_element_type=jnp.float32)
  qrep = qrep.reshape(qrep.shape[0], sl, ll)
  prod = scaled * qrep[:, None, :, :]
  prod = jnp.sum(prod, axis=2)
  w = ll
  while w > vd:
    half = w // 2
    prod = prod[:, :, :half] + prod[:, :, half:w]
    w = half
  ctx_ref[0] = jnp.sum(prod, axis=1)


def _k4_call(staged, tope, sd, tq, *, k_sel, tb, s_off, interpret=False):
  nrows, sl, ll = staged.shape
  kvcols = sl * ll
  nsp, s_tot, _ = tope.shape
  kd = tq.shape[1]
  vd = kvcols // kd
  nt = s_tot // tb
  grid = (nt, nsp)
  staged4 = staged.reshape(nsp * nt, tb, k_sel, sl, ll)
  ns = sd.shape[1]
  full = lambda *shape: pl.BlockSpec(shape, lambda t, s: (0,) * len(shape))
  return pl.pallas_call(
      functools.partial(_k4_body, k_sel=k_sel, tb=tb, kd=kd, vd=vd,
                        s_off=s_off),
      grid=grid,
      in_specs=[
          pl.BlockSpec((1, tb, k_sel, sl, ll),
                       lambda t, s: (s * nt + t, 0, 0, 0, 0)),
          full(nsp, s_tot, k_sel),
          full(s_tot, ns),
          pl.BlockSpec((tb, kd), lambda t, s: (t, 0)),
      ],
      out_specs=pl.BlockSpec((1, tb, vd), lambda t, s: (s, t, 0)),
      out_shape=jax.ShapeDtypeStruct((nsp, s_tot, vd), jnp.float32),
      interpret=interpret,
  )(staged4, tope, sd, tq)


def _k5_body(ca_ref, cb_ref, xe_ref, wo_ref, bo_ref, g_ref, b_ref, out_ref):
  ctx = jnp.sum(ca_ref[...], axis=0) + jnp.sum(cb_ref[...], axis=0)
  proj = lax.dot_general(ctx, wo_ref[...], (((1,), (0,)), ((), ())),
                         preferred_element_type=jnp.float32) + bo_ref[...]
  h = xe_ref[...] + proj
  mean = jnp.mean(h, axis=1, keepdims=True)
  var = jnp.mean((h - mean) * (h - mean), axis=1, keepdims=True)
  out_ref[...] = (h - mean) * lax.rsqrt(var + 1e-5) * g_ref[...] + b_ref[...]


def _k5_call(ctx_a, ctx_b, x_emb, wo, bo, gamma, beta, *, tb,
             interpret=False):
  nsa, s_tot, vd = ctx_a.shape
  nsb = ctx_b.shape[0]
  d = x_emb.shape[1]
  nt = s_tot // tb
  full = lambda *shape: pl.BlockSpec(shape, lambda t: (0,) * len(shape))
  return pl.pallas_call(
      _k5_body,
      grid=(nt,),
      in_specs=[
          pl.BlockSpec((nsa, tb, vd), lambda t: (0, t, 0)),
          pl.BlockSpec((nsb, tb, vd), lambda t: (0, t, 0)),
          pl.BlockSpec((tb, d), lambda t: (t, 0)),
          full(vd, d),
          full(1, d),
          full(1, d),
          full(1, d),
      ],
      out_specs=pl.BlockSpec((tb, d), lambda t: (t, 0)),
      out_shape=jax.ShapeDtypeStruct((s_tot, d), jnp.float32),
      interpret=interpret,
  )(ctx_a, ctx_b, x_emb, wo, bo.reshape(1, d), gamma.reshape(1, d),
    beta.reshape(1, d))


def kernel(x, emb, sector_keys, memory_keys, knowledge, Wq, bq, Wo, bo,
           gamma, beta):
  b, s_tot = x.shape
  ns, m, kd, vd = knowledge.shape
  d = emb.shape[1]
  k_sel = 8
  nsp = ns // 2

  xf = x.reshape(b * s_tot)
  x_emb = _sc_row_gather(emb, xf, rows_per_worker=(b * s_tot) // 32, chunk=32)
  table = knowledge.reshape(ns * m, 8, (kd * vd) // 8)

  ctx_halves = []
  sd = None
  for h in range(2):
    s_off = h * nsp
    sd_h, tq_h, fidx_h, tope_h = _k2_call(
        x_emb, memory_keys[s_off:s_off + nsp], sector_keys, Wq, bq,
        k_sel=k_sel, tb=128, chunk_l=128)
    if h == 0:
      sd, tq = sd_h, tq_h
    offs = ((jnp.arange(nsp, dtype=jnp.int32) + s_off) * m)[:, None, None]
    gidx = (fidx_h + offs).reshape(-1)
    staged = _sc_row_gather(table, gidx,
                            rows_per_worker=gidx.shape[0] // 32, chunk=32)
    ctx_halves.append(_k4_call(staged, tope_h, sd, tq,
                               k_sel=k_sel, tb=256, s_off=s_off))

  out = _k5_call(ctx_halves[0], ctx_halves[1], x_emb, Wo, bo, gamma, beta,
                 tb=256)
  return out.reshape(b, s_tot, d), sd.reshape(b, s_tot, ns)


# speedup vs baseline: 20.3839x; 1.0587x over previous
import functools

import jax
import jax.numpy as jnp
from jax import lax
from jax.experimental import pallas as pl
from jax.experimental.pallas import tpu as pltpu
from jax.experimental.pallas import tpu_sc as plsc

_NEG_INF = float("-inf")
_BIG_I32 = 2**30


def _sc_row_gather(table, idx, rows_per_worker, chunk):
  n = idx.shape[0]
  row_shape = table.shape[1:]
  info = plsc.get_sparse_core_info()
  nc, ns = info.num_cores, info.num_subcores
  nw = nc * ns
  assert n == nw * rows_per_worker
  assert rows_per_worker % chunk == 0
  nchunks = rows_per_worker // chunk
  assert nchunks % 2 == 0 and nchunks >= 2
  mesh = plsc.VectorSubcoreMesh(core_axis_name="c", subcore_axis_name="s")

  @functools.partial(
      pl.kernel,
      mesh=mesh,
      out_type=jax.ShapeDtypeStruct((n,) + row_shape, table.dtype),
      scratch_types=[
          pltpu.VMEM((2, chunk), jnp.int32),
          pltpu.VMEM((2, chunk) + row_shape, table.dtype),
          pltpu.SemaphoreType.DMA((2,)),
          pltpu.SemaphoreType.DMA((2,)),
      ],
  )
  def k(table_hbm, idx_hbm, out_hbm, idx_v, rows_v, gsem, wsem):
    wid = lax.axis_index("s") * nc + lax.axis_index("c")
    wbase = wid * rows_per_worker

    def start_gather(i, p):
      base = wbase + i * chunk
      pltpu.sync_copy(idx_hbm.at[pl.ds(base, chunk)], idx_v.at[p])
      pltpu.async_copy(table_hbm.at[idx_v.at[p]], rows_v.at[p], gsem.at[p])

    def wait_gather(p):
      pltpu.make_async_copy(
          table_hbm.at[idx_v.at[p]], rows_v.at[p], gsem.at[p]).wait()

    def start_write(i, p):
      base = wbase + i * chunk
      pltpu.async_copy(rows_v.at[p], out_hbm.at[pl.ds(base, chunk)],
                       wsem.at[p])

    def wait_write(i, p):
      base = wbase + i * chunk
      pltpu.make_async_copy(
          rows_v.at[p], out_hbm.at[pl.ds(base, chunk)], wsem.at[p]).wait()

    start_gather(0, 0)

    @pl.loop(0, nchunks, step=2)
    def _(i2):
      for b in range(2):
        i = i2 + b
        p = b
        q = 1 - b

        @pl.when(i + 1 < nchunks)
        def _():
          @pl.when(i >= 1)
          def _():
            wait_write(i - 1, q)
          start_gather(i + 1, q)

        wait_gather(p)
        start_write(i, p)

    wait_write(nchunks - 2, 0)
    wait_write(nchunks - 1, 1)

  return k(table, idx)


def _k2_body(xe_ref, mk_ref, sk_ref, wq_ref, bq_ref,
             sd_ref, tq_ref, fidx_ref, tope_ref,
             *, k_sel, chunk_l):
  xe = xe_ref[...]
  mk = mk_ref[0]
  tb = xe.shape[0]
  m = mk.shape[0]
  nchunk = m // chunk_l

  sks = sk_ref[...]
  ss = lax.dot_general(xe, sks, (((1,), (1,)), ((), ())),
                       preferred_element_type=jnp.float32)
  ss = ss - jnp.max(ss, axis=1, keepdims=True)
  es = jnp.exp(ss)
  sd_ref[...] = es / jnp.sum(es, axis=1, keepdims=True)
  tq_ref[...] = lax.dot_general(xe, wq_ref[...], (((1,), (0,)), ((), ())),
                                preferred_element_type=jnp.float32) + bq_ref[...]

  scores = lax.dot_general(xe, mk, (((1,), (1,)), ((), ())),
                           preferred_element_type=jnp.float32)
  sc3 = scores.reshape(tb, nchunk, chunk_l)
  m1 = jnp.max(sc3, axis=2)
  rowmax = jnp.max(m1, axis=1, keepdims=True)

  big_f = float(2**24)
  iota_c = lax.broadcasted_iota(jnp.int32, (tb, nchunk), 1).astype(jnp.float32)
  m1w = m1
  cids = []
  n_csel = min(k_sel, nchunk)
  for _ in range(n_csel):
    v = jnp.max(m1w, axis=1, keepdims=True)
    cid = jnp.min(jnp.where(m1w == v, iota_c, big_f), axis=1)
    cids.append(cid)
    m1w = jnp.where(iota_c == cid[:, None], _NEG_INF, m1w)

  cids_arr = jnp.stack(cids, axis=1)
  iota_c3 = lax.broadcasted_iota(jnp.int32, (tb, n_csel, nchunk),
                                 2).astype(jnp.float32)
  onehot8 = (cids_arr[:, :, None] == iota_c3
             ).astype(jnp.float32)
  cw = lax.dot_general(onehot8, sc3, (((2,), (1,)), ((0,), (0,))),
                       preferred_element_type=jnp.float32)
  iota_l3 = lax.broadcasted_iota(jnp.int32, (tb, n_csel, chunk_l),
                                 2).astype(jnp.float32)
  flatc = cids_arr[:, :, None] * float(chunk_l) + iota_l3

  vals = []
  fidxs = []
  for _ in range(k_sel):
    v = jnp.max(jnp.max(cw, axis=2), axis=1)
    eq = cw == v[:, None, None]
    fi = jnp.min(jnp.min(jnp.where(eq, flatc, big_f), axis=2), axis=1)
    vals.append(v)
    fidxs.append(fi)
    cw = jnp.where(flatc == fi[:, None, None], _NEG_INF, cw)

  vstack = jnp.stack(vals, axis=1)
  fidx_ref[0] = jnp.stack(fidxs, axis=1).astype(jnp.int32)
  tope_ref[0] = jnp.exp(vstack - rowmax)


def _k2_call(x_emb, memory_keys, sector_keys, wq, bq, *, k_sel, tb, chunk_l,
             interpret=False):
  s_tot, d = x_emb.shape
  nsp, m, _ = memory_keys.shape
  ns_all = sector_keys.shape[0]
  nt = s_tot // tb
  grid = (nsp, nt)
  kd = wq.shape[1]
  out_shapes = [
      jax.ShapeDtypeStruct((s_tot, ns_all), jnp.float32),
      jax.ShapeDtypeStruct((s_tot, kd), jnp.float32),
      jax.ShapeDtypeStruct((nsp, s_tot, k_sel), jnp.int32),
      jax.ShapeDtypeStruct((nsp, s_tot, k_sel), jnp.float32),
  ]
  return pl.pallas_call(
      functools.partial(_k2_body, k_sel=k_sel, chunk_l=chunk_l),
      grid=grid,
      in_specs=[
          pl.BlockSpec((tb, d), lambda s, t: (t, 0)),
          pl.BlockSpec((1, m, d), lambda s, t: (s, 0, 0)),
          pl.BlockSpec((ns_all, d), lambda s, t: (0, 0)),
          pl.BlockSpec((d, kd), lambda s, t: (0, 0)),
          pl.BlockSpec((1, kd), lambda s, t: (0, 0)),
      ],
      out_specs=[
          pl.BlockSpec((tb, ns_all), lambda s, t: (t, 0)),
          pl.BlockSpec((tb, kd), lambda s, t: (t, 0)),
          pl.BlockSpec((1, tb, k_sel), lambda s, t: (s, t, 0)),
          pl.BlockSpec((1, tb, k_sel), lambda s, t: (s, t, 0)),
      ],
      out_shape=out_shapes,
      interpret=interpret,
  )(x_emb, memory_keys, sector_keys, wq, bq.reshape(1, kd))


def _k4_body(staged_ref, tope_ref, sd_ref, tq_ref, ctx_ref, acc_ref,
             *, k_sel, tb, kd, vd, s_off, nsp):
  s = pl.program_id(1)
  t = pl.program_id(0)
  tpos = t * tb

  e_blk = tope_ref[s, pl.ds(tpos, tb), :]
  sum_e = jnp.sum(e_blk, axis=1)
  wnorm = e_blk / sum_e[:, None]
  sd_blk = sd_ref[pl.ds(tpos, tb), :]
  iota_s = lax.broadcasted_iota(jnp.int32, sd_blk.shape, 1)
  sd_s = jnp.sum(jnp.where(iota_s == s + s_off, sd_blk, 0.0), axis=1)
  wtilde = wnorm * sd_s[:, None]

  ent = staged_ref[0]
  scaled = ent * wtilde[:, :, None]

  @pl.when(s == 0)
  def _():
    acc_ref[...] = scaled

  @pl.when(s > 0)
  def _():
    acc_ref[...] = acc_ref[...] + scaled

  @pl.when(s == nsp - 1)
  def _():
    q = tq_ref[...]
    io_l = lax.broadcasted_iota(jnp.int32, (kd, kd * vd), 1)
    io_d = lax.broadcasted_iota(jnp.int32, (kd, kd * vd), 0)
    spread = (io_l // vd == io_d).astype(jnp.float32)
    qrep = lax.dot_general(q, spread, (((1,), (0,)), ((), ())),
                           preferred_element_type=jnp.float32)
    prod = acc_ref[...] * qrep[:, None, :]
    w = kd * vd
    while w > vd:
      half = w // 2
      prod = prod[:, :, :half] + prod[:, :, half:w]
      w = half
    ctx_ref[...] = jnp.sum(prod, axis=1)


def _k4_call(staged, tope, sd, tq, *, k_sel, tb, s_off, interpret=False):
  nrows, kvcols = staged.shape
  nsp, s_tot, _ = tope.shape
  kd = tq.shape[1]
  vd = kvcols // kd
  nt = s_tot // tb
  grid = (nt, nsp)
  staged4 = staged.reshape(nsp * nt, tb, k_sel, kvcols)
  ns = sd.shape[1]
  full = lambda *shape: pl.BlockSpec(shape, lambda t, s: (0,) * len(shape))
  return pl.pallas_call(
      functools.partial(_k4_body, k_sel=k_sel, tb=tb, kd=kd, vd=vd,
                        s_off=s_off, nsp=nsp),
      grid=grid,
      in_specs=[
          pl.BlockSpec((1, tb, k_sel, kvcols),
                       lambda t, s: (s * nt + t, 0, 0, 0)),
          full(nsp, s_tot, k_sel),
          full(s_tot, ns),
          pl.BlockSpec((tb, kd), lambda t, s: (t, 0)),
      ],
      out_specs=pl.BlockSpec((tb, vd), lambda t, s: (t, 0)),
      out_shape=jax.ShapeDtypeStruct((s_tot, vd), jnp.float32),
      scratch_shapes=[pltpu.VMEM((tb, k_sel, kvcols), jnp.float32)],
      interpret=interpret,
  )(staged4, tope, sd, tq)


def _k5_body(ca_ref, cb_ref, xe_ref, wo_ref, bo_ref, g_ref, b_ref, out_ref):
  ctx = ca_ref[...] + cb_ref[...]
  proj = lax.dot_general(ctx, wo_ref[...], (((1,), (0,)), ((), ())),
                         preferred_element_type=jnp.float32) + bo_ref[...]
  h = xe_ref[...] + proj
  mean = jnp.mean(h, axis=1, keepdims=True)
  var = jnp.mean((h - mean) * (h - mean), axis=1, keepdims=True)
  out_ref[...] = (h - mean) * lax.rsqrt(var + 1e-5) * g_ref[...] + b_ref[...]


def _k5_call(ctx_a, ctx_b, x_emb, wo, bo, gamma, beta, *, tb,
             interpret=False):
  s_tot, vd = ctx_a.shape
  d = x_emb.shape[1]
  nt = s_tot // tb
  full = lambda *shape: pl.BlockSpec(shape, lambda t: (0,) * len(shape))
  return pl.pallas_call(
      _k5_body,
      grid=(nt,),
      in_specs=[
          pl.BlockSpec((tb, vd), lambda t: (t, 0)),
          pl.BlockSpec((tb, vd), lambda t: (t, 0)),
          pl.BlockSpec((tb, d), lambda t: (t, 0)),
          full(vd, d),
          full(1, d),
          full(1, d),
          full(1, d),
      ],
      out_specs=pl.BlockSpec((tb, d), lambda t: (t, 0)),
      out_shape=jax.ShapeDtypeStruct((s_tot, d), jnp.float32),
      interpret=interpret,
  )(ctx_a, ctx_b, x_emb, wo, bo.reshape(1, d), gamma.reshape(1, d),
    beta.reshape(1, d))


def kernel(x, emb, sector_keys, memory_keys, knowledge, Wq, bq, Wo, bo,
           gamma, beta):
  b, s_tot = x.shape
  ns, m, kd, vd = knowledge.shape
  d = emb.shape[1]
  k_sel = 8
  nsp = ns // 2

  xf = x.reshape(b * s_tot)
  x_emb = _sc_row_gather(emb, xf, rows_per_worker=(b * s_tot) // 32, chunk=32)
  table = knowledge.reshape(ns * m, kd * vd)

  ctx_halves = []
  sd = None
  for h in range(2):
    s_off = h * nsp
    sd_h, tq_h, fidx_h, tope_h = _k2_call(
        x_emb, memory_keys[s_off:s_off + nsp], sector_keys, Wq, bq,
        k_sel=k_sel, tb=128, chunk_l=128)
    if h == 0:
      sd, tq = sd_h, tq_h
    offs = ((jnp.arange(nsp, dtype=jnp.int32) + s_off) * m)[:, None, None]
    gidx = (fidx_h + offs).reshape(-1)
    staged = _sc_row_gather(table, gidx,
                            rows_per_worker=gidx.shape[0] // 32, chunk=32)
    ctx_halves.append(_k4_call(staged, tope_h, sd, tq,
                               k_sel=k_sel, tb=256, s_off=s_off))

  out = _k5_call(ctx_halves[0], ctx_halves[1], x_emb, Wo, bo, gamma, beta,
                 tb=256)
  return out.reshape(b, s_tot, d), sd.reshape(b, s_tot, ns)
